# Initial kernel scaffold; baseline (speedup 1.0000x reference)
import functools

import jax
import jax.numpy as jnp
from jax import lax
from jax.experimental import pallas as pl
from jax.experimental.pallas import tpu as pltpu
from jax.experimental.pallas import tpu_sc as plsc

D = 128
NC = 2
NS = 16
NW = NC * NS
CHUNK = 400


@jax.jit
def _link_dot(x_user, x_item, idx_u, idx_v):
    E = idx_u.shape[0]
    epw = E // NW
    nchunks = epw // CHUNK
    groups = CHUNK // 16

    mesh = plsc.VectorSubcoreMesh(core_axis_name="c", subcore_axis_name="s")

    @functools.partial(
        pl.kernel,
        out_type=jax.ShapeDtypeStruct((E,), jnp.float32),
        mesh=mesh,
        scratch_types=[
            pltpu.VMEM((CHUNK,), jnp.int32),
            pltpu.VMEM((CHUNK,), jnp.int32),
            pltpu.VMEM((CHUNK, D), jnp.float32),
            pltpu.VMEM((CHUNK, D), jnp.float32),
            pltpu.VMEM((CHUNK,), jnp.float32),
            pltpu.SemaphoreType.DMA,
            pltpu.SemaphoreType.DMA,
        ],
    )
    def k(xu_hbm, xi_hbm, iu_hbm, iv_hbm, out_hbm,
          iu_vm, iv_vm, u_rows, v_rows, out_vm, sem_u, sem_v):
        wid = lax.axis_index("s") * NC + lax.axis_index("c")
        base = wid * epw

        @pl.loop(0, nchunks)
        def _chunk(g):
            off = base + g * CHUNK
            pltpu.sync_copy(iu_hbm.at[pl.ds(off, CHUNK)], iu_vm)
            pltpu.sync_copy(iv_hbm.at[pl.ds(off, CHUNK)], iv_vm)
            cu = pltpu.async_copy(xu_hbm.at[iu_vm], u_rows, sem_u)
            cv = pltpu.async_copy(xi_hbm.at[iv_vm], v_rows, sem_v)
            cu.wait()
            cv.wait()

            @pl.loop(0, groups)
            def _group(t):
                rows = t * 16 + lax.iota(jnp.int32, 16)

                def dim_body(d, acc):
                    cols = jnp.full((16,), d, jnp.int32)
                    ug = plsc.load_gather(u_rows, [rows, cols])
                    vg = plsc.load_gather(v_rows, [rows, cols])
                    return acc + ug * vg

                acc = lax.fori_loop(0, D, dim_body,
                                    jnp.zeros((16,), jnp.float32),
                                    unroll=8)
                out_vm[pl.ds(t * 16, 16)] = acc

            pltpu.sync_copy(out_vm, out_hbm.at[pl.ds(off, CHUNK)])

    return k(x_user, x_item, idx_u, idx_v)


def kernel(x_user, x_item, edge_label_index):
    idx = edge_label_index.astype(jnp.int32)
    return _link_dot(x_user, x_item, idx[0], idx[1])


# speedup vs baseline: 1.2040x; 1.2040x over previous
import dataclasses
import functools

import jax
import jax.numpy as jnp
from jax import lax
from jax.experimental import pallas as pl
from jax.experimental.pallas import tpu as pltpu
from jax.experimental.pallas import tpu_sc as plsc

D = 128
NC = 2
NS = 16
NW = NC * NS
CHUNK = 400


@jax.jit
def _link_dot(x_user, x_item, idx_u, idx_v):
    E = idx_u.shape[0]
    epw = E // NW
    nchunks = epw // CHUNK
    groups = CHUNK // 16

    mesh = plsc.VectorSubcoreMesh(core_axis_name="c", subcore_axis_name="s")
    cp = pltpu.CompilerParams()
    if "needs_layout_passes" in pltpu.CompilerParams.__dataclass_fields__:
        cp = dataclasses.replace(cp, needs_layout_passes=False)

    @functools.partial(
        pl.kernel,
        compiler_params=cp,
        out_type=jax.ShapeDtypeStruct((E,), jnp.float32),
        mesh=mesh,
        scratch_types=[
            pltpu.VMEM((CHUNK,), jnp.int32),
            pltpu.VMEM((CHUNK,), jnp.int32),
            pltpu.VMEM((CHUNK, D), jnp.float32),
            pltpu.VMEM((CHUNK, D), jnp.float32),
            pltpu.VMEM((CHUNK,), jnp.float32),
            pltpu.SemaphoreType.DMA,
            pltpu.SemaphoreType.DMA,
        ],
    )
    def k(xu_hbm, xi_hbm, iu_hbm, iv_hbm, out_hbm,
          iu_vm, iv_vm, u_rows, v_rows, out_vm, sem_u, sem_v):
        wid = lax.axis_index("s") * NC + lax.axis_index("c")
        base = wid * epw

        @pl.loop(0, nchunks)
        def _chunk(g):
            off = base + g * CHUNK
            pltpu.sync_copy(iu_hbm.at[pl.ds(off, CHUNK)], iu_vm)
            pltpu.sync_copy(iv_hbm.at[pl.ds(off, CHUNK)], iv_vm)
            cu = pltpu.async_copy(xu_hbm.at[iu_vm], u_rows, sem_u)
            cv = pltpu.async_copy(xi_hbm.at[iv_vm], v_rows, sem_v)
            cu.wait()
            cv.wait()

            @pl.loop(0, groups)
            def _group(t):
                rows = t * 16 + lax.iota(jnp.int32, 16)

                def dim_body(d, acc):
                    cols = jnp.full((16,), d, jnp.int32)
                    ug = plsc.load_gather(u_rows, [rows, cols])
                    vg = plsc.load_gather(v_rows, [rows, cols])
                    return acc + ug * vg

                acc = lax.fori_loop(0, D, dim_body,
                                    jnp.zeros((16,), jnp.float32),
                                    unroll=8)
                out_vm[pl.ds(t * 16, 16)] = acc

            pltpu.sync_copy(out_vm, out_hbm.at[pl.ds(off, CHUNK)])

    return k(x_user, x_item, idx_u, idx_v)


def kernel(x_user, x_item, edge_label_index):
    idx = edge_label_index.astype(jnp.int32)
    return _link_dot(x_user, x_item, idx[0], idx[1])


# speedup vs baseline: 9.3588x; 7.7732x over previous
import dataclasses
import functools

import jax
import jax.numpy as jnp
from jax import lax
from jax.experimental import pallas as pl
from jax.experimental.pallas import tpu as pltpu
from jax.experimental.pallas import tpu_sc as plsc

D = 128
NC = 2
NS = 16
NW = NC * NS
CHUNK = 80


@jax.jit
def _link_dot(x_user, x_item, idx_u, idx_v):
    E = idx_u.shape[0]
    epw = E // NW
    nchunks = epw // CHUNK
    groups = CHUNK // 16

    mesh = plsc.VectorSubcoreMesh(core_axis_name="c", subcore_axis_name="s")
    cp = pltpu.CompilerParams()
    if "needs_layout_passes" in pltpu.CompilerParams.__dataclass_fields__:
        cp = dataclasses.replace(cp, needs_layout_passes=False)

    @functools.partial(
        pl.kernel,
        out_type=jax.ShapeDtypeStruct((E,), jnp.float32),
        mesh=mesh,
        compiler_params=cp,
        scratch_types=[
            pltpu.VMEM((epw,), jnp.int32),
            pltpu.VMEM((epw,), jnp.int32),
            pltpu.VMEM((2, CHUNK, D), jnp.float32),
            pltpu.VMEM((2, CHUNK, D), jnp.float32),
            pltpu.VMEM((epw,), jnp.float32),
            pltpu.SemaphoreType.DMA,
            pltpu.SemaphoreType.DMA,
            pltpu.SemaphoreType.DMA,
            pltpu.SemaphoreType.DMA,
        ],
    )
    def k(xu_hbm, xi_hbm, iu_hbm, iv_hbm, out_hbm,
          iu_all, iv_all, u_bufs, v_bufs, out_all,
          su0, sv0, su1, sv1):
        wid = lax.axis_index("s") * NC + lax.axis_index("c")
        base = wid * epw
        sem_u = (su0, su1)
        sem_v = (sv0, sv1)

        pltpu.sync_copy(iu_hbm.at[pl.ds(base, epw)], iu_all)
        pltpu.sync_copy(iv_hbm.at[pl.ds(base, epw)], iv_all)

        def issue(g, b):
            pltpu.async_copy(
                xu_hbm.at[iu_all.at[pl.ds(g * CHUNK, CHUNK)]],
                u_bufs.at[b], sem_u[b])
            pltpu.async_copy(
                xi_hbm.at[iv_all.at[pl.ds(g * CHUNK, CHUNK)]],
                v_bufs.at[b], sem_v[b])

        def wait(g, b):
            pltpu.make_async_copy(
                xu_hbm.at[iu_all.at[pl.ds(g * CHUNK, CHUNK)]],
                u_bufs.at[b], sem_u[b]).wait()
            pltpu.make_async_copy(
                xi_hbm.at[iv_all.at[pl.ds(g * CHUNK, CHUNK)]],
                v_bufs.at[b], sem_v[b]).wait()

        def compute(g, b):
            ub = u_bufs.at[b]
            vb = v_bufs.at[b]

            @pl.loop(0, groups)
            def _group(t):
                rows = t * 16 + lax.iota(jnp.int32, 16)

                def dim_body(_, carry):
                    acc, cols = carry
                    ug = plsc.load_gather(ub, [rows, cols])
                    vg = plsc.load_gather(vb, [rows, cols])
                    return acc + ug * vg, (cols + 1) & (D - 1)

                acc, _ = lax.fori_loop(
                    0, D, dim_body,
                    (jnp.zeros((16,), jnp.float32), lax.iota(jnp.int32, 16)),
                    unroll=8)
                out_all[pl.ds(g * CHUNK + t * 16, 16)] = acc

        issue(0, 0)

        @pl.loop(0, (nchunks - 1) // 2)
        def _pair(i):
            g = 2 * i
            issue(g + 1, 1)
            wait(g, 0)
            compute(g, 0)
            issue(g + 2, 0)
            wait(g + 1, 1)
            compute(g + 1, 1)

        wait(nchunks - 1, 0)
        compute(nchunks - 1, 0)

        pltpu.sync_copy(out_all, out_hbm.at[pl.ds(base, epw)])

    return k(x_user, x_item, idx_u, idx_v)


def kernel(x_user, x_item, edge_label_index):
    idx = edge_label_index.astype(jnp.int32)
    return _link_dot(x_user, x_item, idx[0], idx[1])
